# baseline (speedup 1.0000x reference)
import functools

import jax
import jax.numpy as jnp
from jax import lax
from jax.experimental import pallas as pl
from jax.experimental.pallas import tpu as pltpu
from jax.experimental.pallas import tpu_sc as plsc

N = 100000
C = 128
L = 16
NW = 32
CH = 128
CG = CH // L
NCH = 25
SPAN = NCH * CH
TBASE = N - CH


def _body(x_hbm, idx_hbm, xt_hbm, it_hbm, out_hbm,
          x0, x1, o0, o1, i0, i1, sq_v,
          sx0, sx1, si0, si1, so0, so1):
    wid = lax.axis_index("s") * 2 + lax.axis_index("c")
    tstart = jnp.minimum(wid * SPAN, ((N - SPAN) // CH) * CH)

    iota16 = lax.iota(jnp.int32, L)
    xbufs = (x0, x1)
    obufs = (o0, o1)
    ibufs = (i0, i1)
    sxs = (sx0, sx1)
    sis = (si0, si1)
    sos = (so0, so1)

    pltpu.sync_copy(x_hbm.at[:, pl.ds(0, CH)], x0)

    @plsc.parallel_loop(0, 64)
    def sq_body(q):
        evec = (q & ~(L - 1)) + jnp.bitwise_and(iota16 + q, L - 1)
        for cg in range(C // L):
            cvec = cg * L + iota16
            col = plsc.load_gather(x0, [cvec, evec])
            plsc.store_scatter(sq_v, [evec, cvec], col * col)

    def start_fetch(k, p):
        rbase = tstart + k * CH
        pltpu.make_async_copy(
            x_hbm.at[:, pl.ds(rbase, CH)], xbufs[p], sxs[p]).start()
        pltpu.make_async_copy(
            idx_hbm.at[pl.ds(rbase, CH)], ibufs[p], sis[p]).start()

    def wait_fetch(k, p):
        rbase = tstart + k * CH
        pltpu.make_async_copy(
            x_hbm.at[:, pl.ds(rbase, CH)], xbufs[p], sxs[p]).wait()
        pltpu.make_async_copy(
            idx_hbm.at[pl.ds(rbase, CH)], ibufs[p], sis[p]).wait()

    def out_copy_to(rbase, p):
        return pltpu.make_async_copy(
            obufs[p], out_hbm.at[pl.ds(rbase, CH), :], sos[p])

    def out_copy(k, p):
        return out_copy_to(tstart + k * CH, p)

    def compute(p):
        x_v, out_v, idx_v = xbufs[p], obufs[p], ibufs[p]

        @plsc.parallel_loop(0, CH)
        def group_body(q):
            rvec = (q & ~(L - 1)) + jnp.bitwise_and(iota16 + q, L - 1)
            ev_rot = jnp.clip(plsc.load_gather(idx_v, [rvec]), 0, 63)
            for cg in range(C // L):
                cvec = cg * L + iota16
                xv = plsc.load_gather(x_v, [cvec, rvec])
                sv = plsc.load_gather(sq_v, [ev_rot, cvec])
                plsc.store_scatter(out_v, [rvec, cvec], xv * sv)

    start_fetch(0, 0)
    start_fetch(1, 1)

    def pair_body(kk, carry):
        for p in range(2):
            k = 2 * kk + p
            wait_fetch(k, p)

            @pl.when(kk >= 1)
            def _():
                out_copy(k - 2, p).wait()

            compute(p)
            out_copy(k, p).start()

            @pl.when(k + 2 < NCH)
            def _():
                start_fetch(k + 2, p)
        return carry

    lax.fori_loop(0, NCH // 2, pair_body, 0)

    k = NCH - 1
    wait_fetch(k, 0)
    out_copy(k - 2, 0).wait()
    compute(0)
    out_copy(k, 0).start()
    out_copy(k - 1, 1).wait()
    out_copy(k, 0).wait()

    @pl.when(wid == NW - 1)
    def _tail():
        pltpu.sync_copy(xt_hbm, x1)
        pltpu.sync_copy(it_hbm, i1)
        compute(1)
        cp = out_copy_to(TBASE, 1)
        cp.start()
        cp.wait()


@jax.jit
def kernel(inp1, inp2):
    idx32 = inp2.reshape(-1).astype(jnp.int32)
    tail_x = lax.slice(inp1, (0, TBASE), (C, N))
    tail_i = lax.slice(idx32, (TBASE,), (N,))
    mesh = plsc.VectorSubcoreMesh(core_axis_name="c", subcore_axis_name="s")
    run = functools.partial(
        pl.kernel,
        mesh=mesh,
        compiler_params=pltpu.CompilerParams(
            use_tc_tiling_on_sc=True,
            needs_layout_passes=False,
            disable_bounds_checks=True,
            disable_semaphore_checks=True,
            skip_device_barrier=True,
        ),
        out_type=jax.ShapeDtypeStruct((N, C), jnp.float32),
        scratch_types=[
            pltpu.VMEM((C, CH), jnp.float32),
            pltpu.VMEM((C, CH), jnp.float32),
            pltpu.VMEM((CH, C), jnp.float32),
            pltpu.VMEM((CH, C), jnp.float32),
            pltpu.VMEM((CH,), jnp.int32),
            pltpu.VMEM((CH,), jnp.int32),
            pltpu.VMEM((64, C), jnp.float32),
            pltpu.SemaphoreType.DMA,
            pltpu.SemaphoreType.DMA,
            pltpu.SemaphoreType.DMA,
            pltpu.SemaphoreType.DMA,
            pltpu.SemaphoreType.DMA,
            pltpu.SemaphoreType.DMA,
        ],
    )(_body)
    return run(inp1, idx32, tail_x, tail_i)


# speedup vs baseline: 1.0007x; 1.0007x over previous
import functools

import jax
import jax.numpy as jnp
from jax import lax
from jax.experimental import pallas as pl
from jax.experimental.pallas import tpu as pltpu
from jax.experimental.pallas import tpu_sc as plsc

N = 100000
C = 128
L = 16
NW = 32
CH = 128
CG = CH // L
NCH = 25
SPAN = NCH * CH
TBASE = N - CH


def _body(x_hbm, idx_hbm, xt_hbm, it_hbm, out_hbm,
          x0, x1, o0, o1, i0, i1, sq_v,
          sx0, sx1, si0, si1, so0, so1):
    wid = lax.axis_index("s") * 2 + lax.axis_index("c")
    tstart = jnp.minimum(wid * SPAN, ((N - SPAN) // CH) * CH)

    iota16 = lax.iota(jnp.int32, L)
    xbufs = (x0, x1)
    obufs = (o0, o1)
    ibufs = (i0, i1)
    sxs = (sx0, sx1)
    sis = (si0, si1)
    sos = (so0, so1)

    pltpu.sync_copy(x_hbm.at[:, pl.ds(0, CH)], x0)

    @plsc.parallel_loop(0, 64)
    def sq_body(q):
        evec = (q & ~(L - 1)) + jnp.bitwise_and(iota16 + q, L - 1)
        for cg in range(C // L):
            cvec = cg * L + iota16
            col = plsc.load_gather(x0, [cvec, evec])
            plsc.store_scatter(sq_v, [evec, cvec], col * col)

    def start_fetch(k, p):
        rbase = tstart + k * CH
        pltpu.make_async_copy(
            x_hbm.at[:, pl.ds(rbase, CH)], xbufs[p], sxs[p]).start()
        pltpu.make_async_copy(
            idx_hbm.at[pl.ds(rbase, CH)], ibufs[p], sis[p]).start()

    def wait_fetch(k, p):
        rbase = tstart + k * CH
        pltpu.make_async_copy(
            x_hbm.at[:, pl.ds(rbase, CH)], xbufs[p], sxs[p]).wait()
        pltpu.make_async_copy(
            idx_hbm.at[pl.ds(rbase, CH)], ibufs[p], sis[p]).wait()

    def out_copy_to(rbase, p):
        return pltpu.make_async_copy(
            obufs[p], out_hbm.at[pl.ds(rbase, CH), :], sos[p])

    def out_copy(k, p):
        return out_copy_to(tstart + k * CH, p)

    def compute(p):
        x_v, out_v, idx_v = xbufs[p], obufs[p], ibufs[p]

        @plsc.parallel_loop(0, CH, unroll=2)
        def group_body(q):
            rvec = (q & ~(L - 1)) + jnp.bitwise_and(iota16 + q, L - 1)
            ev_rot = jnp.clip(plsc.load_gather(idx_v, [rvec]), 0, 63)
            for cg in range(C // L):
                cvec = cg * L + iota16
                xv = plsc.load_gather(x_v, [cvec, rvec])
                sv = plsc.load_gather(sq_v, [ev_rot, cvec])
                plsc.store_scatter(out_v, [rvec, cvec], xv * sv)

    start_fetch(0, 0)
    start_fetch(1, 1)

    def pair_body(kk, carry):
        for p in range(2):
            k = 2 * kk + p
            wait_fetch(k, p)

            @pl.when(kk >= 1)
            def _():
                out_copy(k - 2, p).wait()

            compute(p)
            out_copy(k, p).start()

            @pl.when(k + 2 < NCH)
            def _():
                start_fetch(k + 2, p)
        return carry

    lax.fori_loop(0, NCH // 2, pair_body, 0)

    k = NCH - 1
    wait_fetch(k, 0)
    out_copy(k - 2, 0).wait()
    compute(0)
    out_copy(k, 0).start()
    out_copy(k - 1, 1).wait()
    out_copy(k, 0).wait()

    @pl.when(wid == NW - 1)
    def _tail():
        pltpu.sync_copy(xt_hbm, x1)
        pltpu.sync_copy(it_hbm, i1)
        compute(1)
        cp = out_copy_to(TBASE, 1)
        cp.start()
        cp.wait()


@jax.jit
def kernel(inp1, inp2):
    idx32 = inp2.reshape(-1).astype(jnp.int32)
    tail_x = lax.slice(inp1, (0, TBASE), (C, N))
    tail_i = lax.slice(idx32, (TBASE,), (N,))
    mesh = plsc.VectorSubcoreMesh(core_axis_name="c", subcore_axis_name="s")
    run = functools.partial(
        pl.kernel,
        mesh=mesh,
        compiler_params=pltpu.CompilerParams(
            use_tc_tiling_on_sc=True, needs_layout_passes=False
        ),
        out_type=jax.ShapeDtypeStruct((N, C), jnp.float32),
        scratch_types=[
            pltpu.VMEM((C, CH), jnp.float32),
            pltpu.VMEM((C, CH), jnp.float32),
            pltpu.VMEM((CH, C), jnp.float32),
            pltpu.VMEM((CH, C), jnp.float32),
            pltpu.VMEM((CH,), jnp.int32),
            pltpu.VMEM((CH,), jnp.int32),
            pltpu.VMEM((64, C), jnp.float32),
            pltpu.SemaphoreType.DMA,
            pltpu.SemaphoreType.DMA,
            pltpu.SemaphoreType.DMA,
            pltpu.SemaphoreType.DMA,
            pltpu.SemaphoreType.DMA,
            pltpu.SemaphoreType.DMA,
        ],
    )(_body)
    return run(inp1, idx32, tail_x, tail_i)


# speedup vs baseline: 1.0458x; 1.0450x over previous
import functools

import jax
import jax.numpy as jnp
from jax import lax
from jax.experimental import pallas as pl
from jax.experimental.pallas import tpu as pltpu
from jax.experimental.pallas import tpu_sc as plsc

N = 100000
C = 128
L = 16
NW = 32
CH = 128
CG = CH // L
NCH = 25
SPAN = NCH * CH
TBASE = N - CH


def _body(x_hbm, idx_hbm, xt_hbm, it_hbm, out_hbm,
          x0, x1, x2, x3, o0, o1, i0, i1, i2, i3, sq_v,
          sx0, sx1, sx2, sx3, si0, si1, si2, si3, so0, so1):
    wid = lax.axis_index("s") * 2 + lax.axis_index("c")
    tstart = jnp.minimum(wid * SPAN, ((N - SPAN) // CH) * CH)

    iota16 = lax.iota(jnp.int32, L)
    xbufs = (x0, x1, x2, x3)
    obufs = (o0, o1)
    ibufs = (i0, i1, i2, i3)
    sxs = (sx0, sx1, sx2, sx3)
    sis = (si0, si1, si2, si3)
    sos = (so0, so1)

    pltpu.sync_copy(x_hbm.at[:, pl.ds(0, CH)], x0)

    @plsc.parallel_loop(0, 64)
    def sq_body(q):
        evec = (q & ~(L - 1)) + jnp.bitwise_and(iota16 + q, L - 1)
        for cg in range(C // L):
            cvec = cg * L + iota16
            col = plsc.load_gather(x0, [cvec, evec])
            plsc.store_scatter(sq_v, [evec, cvec], col * col)

    def start_fetch(k, p):
        rbase = tstart + k * CH
        pltpu.make_async_copy(
            x_hbm.at[:, pl.ds(rbase, CH)], xbufs[p], sxs[p]).start()
        pltpu.make_async_copy(
            idx_hbm.at[pl.ds(rbase, CH)], ibufs[p], sis[p]).start()

    def wait_fetch(k, p):
        rbase = tstart + k * CH
        pltpu.make_async_copy(
            x_hbm.at[:, pl.ds(rbase, CH)], xbufs[p], sxs[p]).wait()
        pltpu.make_async_copy(
            idx_hbm.at[pl.ds(rbase, CH)], ibufs[p], sis[p]).wait()

    def out_copy_to(rbase, p):
        return pltpu.make_async_copy(
            obufs[p], out_hbm.at[pl.ds(rbase, CH), :], sos[p])

    def out_copy(k, p):
        return out_copy_to(tstart + k * CH, p)

    def compute(p4, p2):
        x_v, out_v, idx_v = xbufs[p4], obufs[p2], ibufs[p4]

        @plsc.parallel_loop(0, CH, unroll=2)
        def group_body(q):
            rvec = (q & ~(L - 1)) + jnp.bitwise_and(iota16 + q, L - 1)
            ev_rot = jnp.clip(plsc.load_gather(idx_v, [rvec]), 0, 63)
            for cg in range(C // L):
                cvec = cg * L + iota16
                xv = plsc.load_gather(x_v, [cvec, rvec])
                sv = plsc.load_gather(sq_v, [ev_rot, cvec])
                plsc.store_scatter(out_v, [rvec, cvec], xv * sv)

    for k0 in range(4):
        start_fetch(k0, k0)

    def quad_body(kk, carry):
        for p in range(4):
            k = 4 * kk + p
            wait_fetch(k, p)

            @pl.when(k >= 2)
            def _():
                out_copy(k - 2, p % 2).wait()

            compute(p, p % 2)
            out_copy(k, p % 2).start()

            @pl.when(k + 4 < NCH)
            def _():
                start_fetch(k + 4, p)
        return carry

    lax.fori_loop(0, NCH // 4, quad_body, 0)

    k = NCH - 1
    wait_fetch(k, 0)
    out_copy(k - 2, 0).wait()
    compute(0, 0)
    out_copy(k, 0).start()
    out_copy(k - 1, 1).wait()
    out_copy(k, 0).wait()

    @pl.when(wid == NW - 1)
    def _tail():
        pltpu.sync_copy(xt_hbm, x1)
        pltpu.sync_copy(it_hbm, i1)
        compute(1, 1)
        cp = out_copy_to(TBASE, 1)
        cp.start()
        cp.wait()


@jax.jit
def kernel(inp1, inp2):
    idx32 = inp2.reshape(-1).astype(jnp.int32)
    tail_x = lax.slice(inp1, (0, TBASE), (C, N))
    tail_i = lax.slice(idx32, (TBASE,), (N,))
    mesh = plsc.VectorSubcoreMesh(core_axis_name="c", subcore_axis_name="s")
    run = functools.partial(
        pl.kernel,
        mesh=mesh,
        compiler_params=pltpu.CompilerParams(
            use_tc_tiling_on_sc=True, needs_layout_passes=False
        ),
        out_type=jax.ShapeDtypeStruct((N, C), jnp.float32),
        scratch_types=(
            [pltpu.VMEM((C, CH), jnp.float32)] * 4
            + [pltpu.VMEM((CH, C), jnp.float32)] * 2
            + [pltpu.VMEM((CH,), jnp.int32)] * 4
            + [pltpu.VMEM((64, C), jnp.float32)]
            + [pltpu.SemaphoreType.DMA] * 10
        ),
    )(_body)
    return run(inp1, idx32, tail_x, tail_i)
